# baseline (speedup 1.0000x reference)
import functools

import jax
import jax.numpy as jnp
from jax import lax
from jax.experimental import pallas as pl
from jax.experimental.pallas import tpu as pltpu
from jax.experimental.pallas import tpu_sc as plsc

N = 10000
E = 320000
D = 128
H = 128
C = 16

NC = 2
NS = 16
NW = NC * NS

CK = 64
CPP = 40
NBUF = 4
SPT = 4
NSLAB = E // (CPP * CK)
N_PAD = 10240
ROWS_PER_TILE = N_PAD // NS

_mesh = plsc.VectorSubcoreMesh(core_axis_name="c", subcore_axis_name="s")

f32 = jnp.float32
i32 = jnp.int32


@functools.partial(
    pl.kernel,
    out_type=jax.ShapeDtypeStruct((NC, N_PAD), f32),
    mesh=_mesh,
    compiler_params=pltpu.CompilerParams(needs_layout_passes=False),
    scratch_types=[
        pltpu.VMEM((CPP, CK), i32),
        pltpu.VMEM((N_PAD,), f32),
        pltpu.VMEM((N_PAD // NS,), f32),
        pltpu.VMEM((N_PAD // NS,), f32),
        pltpu.VMEM_SHARED((NS, N_PAD), f32),
    ],
)
def _sc_degree(dst_hbm, out_hbm, dstb, hist, accv, tmpv, shist):
    c = lax.axis_index("c")
    s = lax.axis_index("s")
    wid = c * NS + s
    zero16 = jnp.zeros((16,), f32)

    def zrow(i, _):
        for j in range(8):
            hist[pl.ds(i * 128 + j * 16, 16)] = zero16
        return 0

    lax.fori_loop(0, N_PAD // 128, zrow, 0)

    ones = jnp.full((16,), 1.0, f32)

    def acc(ci, _):
        for j in range(CK // 16):
            idx = dstb[ci, pl.ds(j * 16, 16)]
            plsc.addupdate_scatter(hist, [idx], ones)
        return 0

    for p in range(SPT):
        slab = wid * SPT + p

        @pl.when(slab < NSLAB)
        def _():
            pltpu.sync_copy(dst_hbm.at[slab], dstb)
            lax.fori_loop(0, CPP, acc, 0)

    pltpu.sync_copy(hist, shist.at[s])
    plsc.subcore_barrier()

    seg = N_PAD // NS
    base = s * seg
    pltpu.sync_copy(shist.at[0, pl.ds(base, seg)], accv)
    for t in range(1, NS):
        pltpu.sync_copy(shist.at[t, pl.ds(base, seg)], tmpv)

        def add(i, _):
            accv[pl.ds(i * 16, 16)] = (accv[pl.ds(i * 16, 16)]
                                       + tmpv[pl.ds(i * 16, 16)])
            return 0

        lax.fori_loop(0, seg // 16, add, 0)
    pltpu.sync_copy(accv, out_hbm.at[c, pl.ds(base, seg)])


@functools.partial(
    pl.kernel,
    out_type=jax.ShapeDtypeStruct((NC, N_PAD, 128), f32),
    mesh=_mesh,
    scratch_types=(
        [pltpu.VMEM((CPP, CK), i32),
         pltpu.VMEM((CPP, CK), i32)]
        + [pltpu.VMEM((CK, 128), f32) for _ in range(NBUF)]
        + [pltpu.VMEM_SHARED((N_PAD, 128), f32)]
        + [pltpu.SemaphoreType.DMA for _ in range(2 * NBUF)]
    ),
)
def _sc_scatter(hs_hbm, src_hbm, dst_hbm, out_hbm, srcb, dstb, *rest):
    bufs = rest[:NBUF]
    accsh = rest[NBUF]
    semg = rest[NBUF + 1:NBUF + 1 + NBUF]
    sems = rest[NBUF + 1 + NBUF:]
    c = lax.axis_index("c")
    s = lax.axis_index("s")
    zero16 = jnp.zeros((16,), f32)
    r0 = bufs[0]

    def zrow(i, _):
        for j in range(8):
            r0[i, pl.ds(j * 16, 16)] = zero16
        return 0

    lax.fori_loop(0, CK, zrow, 0)

    def zsh(i, _):
        pltpu.sync_copy(r0, accsh.at[pl.ds(s * ROWS_PER_TILE + i * CK, CK)])
        return 0

    lax.fori_loop(0, ROWS_PER_TILE // CK, zsh, 0)
    plsc.subcore_barrier()

    def gstart(ci, b):
        pltpu.make_async_copy(hs_hbm.at[srcb.at[ci]], bufs[b], semg[b]).start()

    def gwait(ci, b):
        pltpu.make_async_copy(hs_hbm.at[srcb.at[ci]], bufs[b], semg[b]).wait()

    def sstart(ci, b):
        pltpu.async_copy(bufs[b], accsh.at[dstb.at[ci]], sems[b], add=True)

    def swait(ci, b):
        pltpu.make_async_copy(bufs[b], accsh.at[dstb.at[ci]], sems[b]).wait()

    def do_slab(slab):
        pltpu.sync_copy(src_hbm.at[slab], srcb)
        pltpu.sync_copy(dst_hbm.at[slab], dstb)
        for b in range(NBUF):
            gstart(b, b)

        def group(i, _):
            c0 = NBUF * i
            for b in range(NBUF):
                gwait(c0 + b, b)
                sstart(c0 + b, b)
            for b in range(NBUF):
                swait(c0 + b, b)
                gstart(c0 + NBUF + b, b)
            return 0

        lax.fori_loop(0, CPP // NBUF - 1, group, 0)

        last = CPP - NBUF
        for b in range(NBUF):
            gwait(last + b, b)
            sstart(last + b, b)
        for b in range(NBUF):
            swait(last + b, b)

    wid = c * NS + s
    for p in range(SPT):
        slab = wid * SPT + p

        @pl.when(slab < NSLAB)
        def _():
            do_slab(slab)

    plsc.subcore_barrier()
    pltpu.sync_copy(accsh.at[pl.ds(s * ROWS_PER_TILE, ROWS_PER_TILE)],
                    out_hbm.at[c, pl.ds(s * ROWS_PER_TILE, ROWS_PER_TILE)])


RB = 2000
GRID = N // RB


def _tc1a_body(x_ref, w_ref, hw_ref):
    hw_ref[...] = jnp.dot(x_ref[...], w_ref[...],
                          preferred_element_type=f32,
                          precision=lax.Precision.HIGHEST)


def _tc1a(x, w1):
    return pl.pallas_call(
        _tc1a_body,
        grid=(GRID,),
        in_specs=[
            pl.BlockSpec((RB, D), lambda i: (i, 0)),
            pl.BlockSpec((D, H), lambda i: (0, 0)),
        ],
        out_specs=pl.BlockSpec((RB, H), lambda i: (i, 0)),
        out_shape=jax.ShapeDtypeStruct((N, H), f32),
    )(x, w1)


def _tc1b_body(hw_ref, hist_ref, hs_ref, dinv_ref):
    deg = jnp.sum(hist_ref[...], axis=0) + 1.0
    y = lax.rsqrt(deg)
    dinv = y * (1.5 - 0.5 * deg * y * y)
    hs_ref[...] = hw_ref[...] * dinv
    dinv_ref[...] = dinv


def _tc1b(hw, hist_r):
    return pl.pallas_call(
        _tc1b_body,
        grid=(GRID,),
        in_specs=[
            pl.BlockSpec((RB, H), lambda i: (i, 0)),
            pl.BlockSpec((NC, RB, 1), lambda i: (0, i, 0)),
        ],
        out_specs=[
            pl.BlockSpec((RB, H), lambda i: (i, 0)),
            pl.BlockSpec((RB, 1), lambda i: (i, 0)),
        ],
        out_shape=[
            jax.ShapeDtypeStruct((N, H), f32),
            jax.ShapeDtypeStruct((N, 1), f32),
        ],
    )(hw, hist_r)


def _tc_mid_body(p_ref, hsp_ref, dinv_ref, b_ref, w_ref, out_ref):
    acc = p_ref[0] + p_ref[1] + hsp_ref[...]
    h = jnp.maximum(dinv_ref[...] * acc + b_ref[...], 0.0)
    out_ref[...] = jnp.dot(h, w_ref[...], preferred_element_type=f32, precision=lax.Precision.HIGHEST) * dinv_ref[...]


def _tc_mid(p, hs_prev, dinv, b_row, w_next):
    return pl.pallas_call(
        _tc_mid_body,
        grid=(GRID,),
        in_specs=[
            pl.BlockSpec((NC, RB, H), lambda i: (0, i, 0)),
            pl.BlockSpec((RB, H), lambda i: (i, 0)),
            pl.BlockSpec((RB, 1), lambda i: (i, 0)),
            pl.BlockSpec((1, H), lambda i: (0, 0)),
            pl.BlockSpec((H, H), lambda i: (0, 0)),
        ],
        out_specs=pl.BlockSpec((RB, H), lambda i: (i, 0)),
        out_shape=jax.ShapeDtypeStruct((N, H), f32),
    )(p, hs_prev, dinv, b_row, w_next)


def _tc_final_body(p_ref, hsp_ref, dinv_ref, b_ref, wc_ref, bc_ref,
                   wg1_ref, bg1_ref, wg2_ref, bg2_ref,
                   logits_ref, gpred_ref, emb_ref, colsum):
    i = pl.program_id(0)
    acc = p_ref[0] + p_ref[1] + hsp_ref[...]
    emb = dinv_ref[...] * acc + b_ref[...]
    emb_ref[...] = emb
    logits_ref[...] = jnp.dot(emb, wc_ref[...],
                              preferred_element_type=f32, precision=lax.Precision.HIGHEST) + bc_ref[...]
    part = jnp.sum(emb, axis=0, keepdims=True)

    @pl.when(i == 0)
    def _():
        colsum[...] = part

    @pl.when(i > 0)
    def _():
        colsum[...] += part

    @pl.when(i == GRID - 1)
    def _():
        ge = colsum[...] * (1.0 / N)
        g = jnp.maximum(jnp.dot(ge, wg1_ref[...],
                                preferred_element_type=f32, precision=lax.Precision.HIGHEST) + bg1_ref[...], 0.0)
        gpred_ref[...] = jnp.dot(g, wg2_ref[...],
                                 preferred_element_type=f32, precision=lax.Precision.HIGHEST) + bg2_ref[...]


def _tc_final(p, hs3, dinv, b_row, wc, bc_row, wg1, bg1_row, wg2, bg2_row):
    return pl.pallas_call(
        _tc_final_body,
        grid=(GRID,),
        in_specs=[
            pl.BlockSpec((NC, RB, H), lambda i: (0, i, 0)),
            pl.BlockSpec((RB, H), lambda i: (i, 0)),
            pl.BlockSpec((RB, 1), lambda i: (i, 0)),
            pl.BlockSpec((1, H), lambda i: (0, 0)),
            pl.BlockSpec((H, C), lambda i: (0, 0)),
            pl.BlockSpec((1, C), lambda i: (0, 0)),
            pl.BlockSpec((H, H // 2), lambda i: (0, 0)),
            pl.BlockSpec((1, H // 2), lambda i: (0, 0)),
            pl.BlockSpec((H // 2, 1), lambda i: (0, 0)),
            pl.BlockSpec((1, 1), lambda i: (0, 0)),
        ],
        out_specs=[
            pl.BlockSpec((RB, C), lambda i: (i, 0)),
            pl.BlockSpec((1, 1), lambda i: (0, 0)),
            pl.BlockSpec((RB, H), lambda i: (i, 0)),
        ],
        out_shape=[
            jax.ShapeDtypeStruct((N, C), f32),
            jax.ShapeDtypeStruct((1, 1), f32),
            jax.ShapeDtypeStruct((N, H), f32),
        ],
        scratch_shapes=[pltpu.VMEM((1, H), f32)],
    )(p, hs3, dinv, b_row, wc, bc_row, wg1, bg1_row, wg2, bg2_row)


def kernel(x, edge_index, W1, b1, W2, b2, W3, b3, Wc, bc, Wg1, bg1, Wg2, bg2):
    src4 = edge_index[0].reshape(NSLAB, CPP, CK)
    dst4 = edge_index[1].reshape(NSLAB, CPP, CK)

    hist = _sc_degree(dst4)
    hist_r = hist.reshape(NC, N_PAD, 1)

    hw1 = _tc1a(x, W1)
    hs1, dinv = _tc1b(hw1, hist_r)
    p1 = _sc_scatter(hs1, src4, dst4)
    hs2 = _tc_mid(p1, hs1, dinv, b1.reshape(1, H), W2)
    p2 = _sc_scatter(hs2, src4, dst4)
    hs3 = _tc_mid(p2, hs2, dinv, b2.reshape(1, H), W3)
    p3 = _sc_scatter(hs3, src4, dst4)
    logits, gpred, emb = _tc_final(
        p3, hs3, dinv, b3.reshape(1, H), Wc, bc.reshape(1, C),
        Wg1, bg1.reshape(1, H // 2), Wg2, bg2.reshape(1, 1))
    return (logits, gpred, emb)


# speedup vs baseline: 1.0235x; 1.0235x over previous
import functools

import jax
import jax.numpy as jnp
from jax import lax
from jax.experimental import pallas as pl
from jax.experimental.pallas import tpu as pltpu
from jax.experimental.pallas import tpu_sc as plsc

N = 10000
E = 320000
D = 128
H = 128
C = 16

NC = 2
NS = 16
NW = NC * NS

CK = 64
CPP = 40
NBUF = 4
SPT = 4
NSLAB = E // (CPP * CK)
N_PAD = 10240
ROWS_PER_TILE = N_PAD // NS

_mesh = plsc.VectorSubcoreMesh(core_axis_name="c", subcore_axis_name="s")

f32 = jnp.float32
i32 = jnp.int32


@functools.partial(
    pl.kernel,
    out_type=jax.ShapeDtypeStruct((NC, N_PAD), f32),
    mesh=_mesh,
    compiler_params=pltpu.CompilerParams(needs_layout_passes=False),
    scratch_types=[
        pltpu.VMEM((CPP, CK), i32),
        pltpu.VMEM((N_PAD,), f32),
        pltpu.VMEM((N_PAD // NS,), f32),
        pltpu.VMEM((N_PAD // NS,), f32),
        pltpu.VMEM_SHARED((NS, N_PAD), f32),
    ],
)
def _sc_degree(eidx_hbm, out_hbm, dstb, hist, accv, tmpv, shist):
    c = lax.axis_index("c")
    s = lax.axis_index("s")
    wid = c * NS + s
    zero16 = jnp.zeros((16,), f32)

    def zrow(i, _):
        for j in range(8):
            hist[pl.ds(i * 128 + j * 16, 16)] = zero16
        return 0

    lax.fori_loop(0, N_PAD // 128, zrow, 0)

    ones = jnp.full((16,), 1.0, f32)

    def acc(ci, _):
        for j in range(CK // 16):
            idx = dstb[ci, pl.ds(j * 16, 16)]
            plsc.addupdate_scatter(hist, [idx], ones)
        return 0

    for p in range(SPT):
        slab = wid * SPT + p

        @pl.when(slab < NSLAB)
        def _():
            pltpu.sync_copy(eidx_hbm.at[1, slab], dstb)
            lax.fori_loop(0, CPP, acc, 0)

    pltpu.sync_copy(hist, shist.at[s])
    plsc.subcore_barrier()

    seg = N_PAD // NS
    base = s * seg
    pltpu.sync_copy(shist.at[0, pl.ds(base, seg)], accv)
    for t in range(1, NS):
        pltpu.sync_copy(shist.at[t, pl.ds(base, seg)], tmpv)

        def add(i, _):
            accv[pl.ds(i * 16, 16)] = (accv[pl.ds(i * 16, 16)]
                                       + tmpv[pl.ds(i * 16, 16)])
            return 0

        lax.fori_loop(0, seg // 16, add, 0)
    pltpu.sync_copy(accv, out_hbm.at[c, pl.ds(base, seg)])


@functools.partial(
    pl.kernel,
    out_type=jax.ShapeDtypeStruct((NC, N_PAD, 128), f32),
    mesh=_mesh,
    scratch_types=(
        [pltpu.VMEM((CPP, CK), i32),
         pltpu.VMEM((CPP, CK), i32)]
        + [pltpu.VMEM((CK, 128), f32) for _ in range(NBUF)]
        + [pltpu.VMEM_SHARED((N_PAD, 128), f32)]
        + [pltpu.SemaphoreType.DMA for _ in range(2 * NBUF)]
    ),
)
def _sc_scatter(hs_hbm, eidx_hbm, out_hbm, srcb, dstb, *rest):
    bufs = rest[:NBUF]
    accsh = rest[NBUF]
    semg = rest[NBUF + 1:NBUF + 1 + NBUF]
    sems = rest[NBUF + 1 + NBUF:]
    c = lax.axis_index("c")
    s = lax.axis_index("s")
    zero16 = jnp.zeros((16,), f32)
    r0 = bufs[0]

    def zrow(i, _):
        for j in range(8):
            r0[i, pl.ds(j * 16, 16)] = zero16
        return 0

    lax.fori_loop(0, CK, zrow, 0)

    def zsh(i, _):
        pltpu.sync_copy(r0, accsh.at[pl.ds(s * ROWS_PER_TILE + i * CK, CK)])
        return 0

    lax.fori_loop(0, ROWS_PER_TILE // CK, zsh, 0)
    plsc.subcore_barrier()

    def gstart(ci, b):
        pltpu.make_async_copy(hs_hbm.at[srcb.at[ci]], bufs[b], semg[b]).start()

    def gwait(ci, b):
        pltpu.make_async_copy(hs_hbm.at[srcb.at[ci]], bufs[b], semg[b]).wait()

    def sstart(ci, b):
        pltpu.async_copy(bufs[b], accsh.at[dstb.at[ci]], sems[b], add=True)

    def swait(ci, b):
        pltpu.make_async_copy(bufs[b], accsh.at[dstb.at[ci]], sems[b]).wait()

    def do_slab(slab):
        pltpu.sync_copy(eidx_hbm.at[0, slab], srcb)
        pltpu.sync_copy(eidx_hbm.at[1, slab], dstb)
        for b in range(NBUF):
            gstart(b, b)

        def group(i, _):
            c0 = NBUF * i
            for b in range(NBUF):
                gwait(c0 + b, b)
                sstart(c0 + b, b)
            for b in range(NBUF):
                swait(c0 + b, b)
                gstart(c0 + NBUF + b, b)
            return 0

        lax.fori_loop(0, CPP // NBUF - 1, group, 0)

        last = CPP - NBUF
        for b in range(NBUF):
            gwait(last + b, b)
            sstart(last + b, b)
        for b in range(NBUF):
            swait(last + b, b)

    wid = c * NS + s
    for p in range(SPT):
        slab = wid * SPT + p

        @pl.when(slab < NSLAB)
        def _():
            do_slab(slab)

    plsc.subcore_barrier()
    pltpu.sync_copy(accsh.at[pl.ds(s * ROWS_PER_TILE, ROWS_PER_TILE)],
                    out_hbm.at[c, pl.ds(s * ROWS_PER_TILE, ROWS_PER_TILE)])


RB = 2000
GRID = N // RB


def _tc1a_body(x_ref, w_ref, hw_ref):
    hw_ref[...] = jnp.dot(x_ref[...], w_ref[...],
                          preferred_element_type=f32,
                          precision=lax.Precision.HIGHEST)


def _tc1a(x, w1):
    return pl.pallas_call(
        _tc1a_body,
        grid=(GRID,),
        in_specs=[
            pl.BlockSpec((RB, D), lambda i: (i, 0)),
            pl.BlockSpec((D, H), lambda i: (0, 0)),
        ],
        out_specs=pl.BlockSpec((RB, H), lambda i: (i, 0)),
        out_shape=jax.ShapeDtypeStruct((N, H), f32),
    )(x, w1)


def _tc1b_body(hw_ref, hist_ref, hs_ref, dinv_ref):
    deg = jnp.sum(hist_ref[...], axis=0) + 1.0
    y = lax.rsqrt(deg)
    dinv = y * (1.5 - 0.5 * deg * y * y)
    hs_ref[...] = hw_ref[...] * dinv
    dinv_ref[...] = dinv


def _tc1b(hw, hist_r):
    return pl.pallas_call(
        _tc1b_body,
        grid=(GRID,),
        in_specs=[
            pl.BlockSpec((RB, H), lambda i: (i, 0)),
            pl.BlockSpec((NC, RB, 1), lambda i: (0, i, 0)),
        ],
        out_specs=[
            pl.BlockSpec((RB, H), lambda i: (i, 0)),
            pl.BlockSpec((RB, 1), lambda i: (i, 0)),
        ],
        out_shape=[
            jax.ShapeDtypeStruct((N, H), f32),
            jax.ShapeDtypeStruct((N, 1), f32),
        ],
    )(hw, hist_r)


def _tc_mid_body(p_ref, hsp_ref, dinv_ref, b_ref, w_ref, out_ref):
    acc = p_ref[0] + p_ref[1] + hsp_ref[...]
    h = jnp.maximum(dinv_ref[...] * acc + b_ref[...], 0.0)
    out_ref[...] = jnp.dot(h, w_ref[...], preferred_element_type=f32, precision=lax.Precision.HIGHEST) * dinv_ref[...]


def _tc_mid(p, hs_prev, dinv, b_row, w_next):
    return pl.pallas_call(
        _tc_mid_body,
        grid=(GRID,),
        in_specs=[
            pl.BlockSpec((NC, RB, H), lambda i: (0, i, 0)),
            pl.BlockSpec((RB, H), lambda i: (i, 0)),
            pl.BlockSpec((RB, 1), lambda i: (i, 0)),
            pl.BlockSpec((1, H), lambda i: (0, 0)),
            pl.BlockSpec((H, H), lambda i: (0, 0)),
        ],
        out_specs=pl.BlockSpec((RB, H), lambda i: (i, 0)),
        out_shape=jax.ShapeDtypeStruct((N, H), f32),
    )(p, hs_prev, dinv, b_row, w_next)


def _tc_final_body(p_ref, hsp_ref, dinv_ref, b_ref, wc_ref, bc_ref,
                   wg1_ref, bg1_ref, wg2_ref, bg2_ref,
                   logits_ref, gpred_ref, emb_ref, colsum):
    i = pl.program_id(0)
    acc = p_ref[0] + p_ref[1] + hsp_ref[...]
    emb = dinv_ref[...] * acc + b_ref[...]
    emb_ref[...] = emb
    logits_ref[...] = jnp.dot(emb, wc_ref[...],
                              preferred_element_type=f32, precision=lax.Precision.HIGHEST) + bc_ref[...]
    part = jnp.sum(emb, axis=0, keepdims=True)

    @pl.when(i == 0)
    def _():
        colsum[...] = part

    @pl.when(i > 0)
    def _():
        colsum[...] += part

    @pl.when(i == GRID - 1)
    def _():
        ge = colsum[...] * (1.0 / N)
        g = jnp.maximum(jnp.dot(ge, wg1_ref[...],
                                preferred_element_type=f32, precision=lax.Precision.HIGHEST) + bg1_ref[...], 0.0)
        gpred_ref[...] = jnp.dot(g, wg2_ref[...],
                                 preferred_element_type=f32, precision=lax.Precision.HIGHEST) + bg2_ref[...]


def _tc_final(p, hs3, dinv, b_row, wc, bc_row, wg1, bg1_row, wg2, bg2_row):
    return pl.pallas_call(
        _tc_final_body,
        grid=(GRID,),
        in_specs=[
            pl.BlockSpec((NC, RB, H), lambda i: (0, i, 0)),
            pl.BlockSpec((RB, H), lambda i: (i, 0)),
            pl.BlockSpec((RB, 1), lambda i: (i, 0)),
            pl.BlockSpec((1, H), lambda i: (0, 0)),
            pl.BlockSpec((H, C), lambda i: (0, 0)),
            pl.BlockSpec((1, C), lambda i: (0, 0)),
            pl.BlockSpec((H, H // 2), lambda i: (0, 0)),
            pl.BlockSpec((1, H // 2), lambda i: (0, 0)),
            pl.BlockSpec((H // 2, 1), lambda i: (0, 0)),
            pl.BlockSpec((1, 1), lambda i: (0, 0)),
        ],
        out_specs=[
            pl.BlockSpec((RB, C), lambda i: (i, 0)),
            pl.BlockSpec((1, 1), lambda i: (0, 0)),
            pl.BlockSpec((RB, H), lambda i: (i, 0)),
        ],
        out_shape=[
            jax.ShapeDtypeStruct((N, C), f32),
            jax.ShapeDtypeStruct((1, 1), f32),
            jax.ShapeDtypeStruct((N, H), f32),
        ],
        scratch_shapes=[pltpu.VMEM((1, H), f32)],
    )(p, hs3, dinv, b_row, wc, bc_row, wg1, bg1_row, wg2, bg2_row)


def kernel(x, edge_index, W1, b1, W2, b2, W3, b3, Wc, bc, Wg1, bg1, Wg2, bg2):
    eidx4 = edge_index.reshape(2, NSLAB, CPP, CK)

    hist = _sc_degree(eidx4)
    hist_r = hist.reshape(NC, N_PAD, 1)

    hw1 = _tc1a(x, W1)
    hs1, dinv = _tc1b(hw1, hist_r)
    p1 = _sc_scatter(hs1, eidx4)
    hs2 = _tc_mid(p1, hs1, dinv, b1.reshape(1, H), W2)
    p2 = _sc_scatter(hs2, eidx4)
    hs3 = _tc_mid(p2, hs2, dinv, b2.reshape(1, H), W3)
    p3 = _sc_scatter(hs3, eidx4)
    logits, gpred, emb = _tc_final(
        p3, hs3, dinv, b3.reshape(1, H), Wc, bc.reshape(1, C),
        Wg1, bg1.reshape(1, H // 2), Wg2, bg2.reshape(1, 1))
    return (logits, gpred, emb)
